# baseline (speedup 1.0000x reference)
import functools

import jax
import jax.numpy as jnp
from jax import lax
from jax.experimental import pallas as pl
from jax.experimental.pallas import tpu as pltpu
from jax.experimental.pallas import tpu_sc as plsc

NC, NS, L = 2, 16, 16
NW = NC * NS
C = 32
NBUF = 2


def _lane_sum(x):
    return jnp.full((L,), jnp.sum(x), dtype=jnp.float32)


def _swz(x):
    shp = x.shape
    nb = shp[-1] // (2 * L)
    y = x.reshape(*shp[:-1], nb, 2, L).swapaxes(-1, -2)
    ybf = y.reshape(*shp[:-1], shp[-1] // 2, 2).astype(jnp.bfloat16)
    return lax.bitcast_convert_type(ybf, jnp.int32)


def _unpk(v32):
    vbf = plsc.bitcast(v32, jnp.bfloat16)
    return plsc.unpack(vbf, format=plsc.PackFormat.INTERLEAVED,
                       preferred_element_type=jnp.float32)


def _sc_body(S, H, TPW, ids_hbm, tts_hbm, word_hbm, pos_hbm, type_hbm,
             gam_hbm, bet_hbm, out_hbm,
             idx_v, ttv, rows_v, pos_v, type_v, gam_v, bet_v,
             gsem0, gsem1, psem0, psem1, osem0, osem1):
    nh2 = H // (2 * L)
    nch = TPW // C
    gsem = (gsem0, gsem1)
    psem = (psem0, psem1)
    osem = (osem0, osem1)
    wid = lax.axis_index("c") * NS + lax.axis_index("s")
    tok0 = wid * TPW
    s0 = lax.rem(tok0, S)

    pltpu.sync_copy(type_hbm, type_v)
    pltpu.sync_copy(gam_hbm, gam_v)
    pltpu.sync_copy(bet_hbm, bet_v)

    inv_h = jnp.float32(1.0 / H)

    def issue_in(c, b):
        base = tok0 + c * C
        pltpu.sync_copy(ids_hbm.at[pl.ds(base, C)], idx_v.at[b])
        pltpu.sync_copy(tts_hbm.at[pl.ds(base, C)], ttv.at[b, pl.ds(0, C)])
        pltpu.async_copy(pos_hbm.at[pl.ds(s0 + c * C, C)], pos_v.at[b],
                         psem[b])
        pltpu.async_copy(word_hbm.at[idx_v.at[b]], rows_v.at[b], gsem[b])

    def wait_in(c, b):
        pltpu.make_async_copy(word_hbm.at[idx_v.at[b]], rows_v.at[b],
                              gsem[b]).wait()
        pltpu.make_async_copy(pos_hbm.at[pl.ds(s0 + c * C, C)], pos_v.at[b],
                              psem[b]).wait()

    def start_out(c, b):
        base = tok0 + c * C
        pltpu.async_copy(rows_v.at[b], out_hbm.at[pl.ds(base, C)], osem[b])

    def wait_out(c, b):
        base = tok0 + c * C
        pltpu.make_async_copy(rows_v.at[b], out_hbm.at[pl.ds(base, C)],
                              osem[b]).wait()

    def compute(c, b):
        @plsc.parallel_loop(0, C, 1)
        def tok_body(j):
            ttvec = ttv[b, pl.ds(j, L)]
            tmask = ttvec[0] != 0
            nacc = 8
            acc_s = [jnp.zeros((L,), jnp.float32) for _ in range(nacc)]
            acc_q = [jnp.zeros((L,), jnp.float32) for _ in range(nacc)]
            for h in range(nh2):
                lo = pl.ds(2 * h * L, L)
                hi = pl.ds((2 * h + 1) * L, L)
                d32 = pl.ds(h * L, L)
                p_lo, p_hi = _unpk(pos_v[b, j, d32])
                t_lo, t_hi = _unpk(jnp.where(tmask, type_v[1, d32],
                                             type_v[0, d32]))
                x_lo = rows_v[b, j, lo] + p_lo + t_lo
                x_hi = rows_v[b, j, hi] + p_hi + t_hi
                k = (2 * h) % nacc
                k2 = (2 * h + 1) % nacc
                acc_s[k] = acc_s[k] + x_lo
                acc_q[k] = acc_q[k] + x_lo * x_lo
                acc_s[k2] = acc_s[k2] + x_hi
                acc_q[k2] = acc_q[k2] + x_hi * x_hi
                xbf = plsc.pack(x_lo, x_hi, format=plsc.PackFormat.INTERLEAVED,
                                preferred_element_type=jnp.bfloat16)
                pos_v[b, j, d32] = plsc.bitcast(xbf, jnp.int32)
            while len(acc_s) > 1:
                acc_s = [p + q for p, q in zip(acc_s[::2], acc_s[1::2])]
                acc_q = [p + q for p, q in zip(acc_q[::2], acc_q[1::2])]
            mean = _lane_sum(acc_s[0]) * inv_h
            var = _lane_sum(acc_q[0]) * inv_h - mean * mean
            v16 = var + jnp.float32(1e-12)
            bits = plsc.bitcast(v16, jnp.int32)
            bits = jnp.int32(0x5F3759DF) - (bits >> 1)
            y = plsc.bitcast(bits, jnp.float32)
            for _ in range(3):
                y = y * (jnp.float32(1.5) - jnp.float32(0.5) * v16 * y * y)
            a = y
            bb = -mean * y
            for h in range(nh2):
                lo = pl.ds(2 * h * L, L)
                hi = pl.ds((2 * h + 1) * L, L)
                d32 = pl.ds(h * L, L)
                g_lo, g_hi = _unpk(gam_v[d32])
                be_lo, be_hi = _unpk(bet_v[d32])
                x_lo, x_hi = _unpk(pos_v[b, j, d32])
                rows_v[b, j, lo] = (x_lo * a + bb) * g_lo + be_lo
                rows_v[b, j, hi] = (x_hi * a + bb) * g_hi + be_hi

    issue_in(0, 0)

    def pair_body(cp, carry):
        for b in range(NBUF):
            c = cp * NBUF + b

            @pl.when(c + 1 < nch)
            def _prefetch():
                @pl.when(c >= 1)
                def _drain():
                    wait_out(c - 1, 1 - b)
                issue_in(c + 1, 1 - b)

            wait_in(c, b)
            compute(c, b)
            start_out(c, b)
        return carry

    lax.fori_loop(0, nch // NBUF, pair_body, 0)
    wait_out(nch - 2, 0)
    wait_out(nch - 1, 1)


def kernel(input_ids, token_type_ids, word_emb, pos_emb, type_emb,
           ln_gamma, ln_beta):
    B, S = input_ids.shape
    H = word_emb.shape[1]
    TOK = B * S
    TPW = TOK // NW

    ids = input_ids.reshape(-1).astype(jnp.int32)
    tts = token_type_ids.reshape(-1).astype(jnp.int32)
    pos_b = _swz(pos_emb)
    type_b = _swz(type_emb)
    gam_b = _swz(ln_gamma)
    bet_b = _swz(ln_beta)

    mesh = plsc.VectorSubcoreMesh(
        core_axis_name="c", subcore_axis_name="s",
        num_cores=NC, num_subcores=NS)
    f = pl.kernel(
        functools.partial(_sc_body, S, H, TPW),
        out_type=jax.ShapeDtypeStruct((TOK, H), jnp.float32),
        mesh=mesh,
        scratch_types=[
            pltpu.VMEM((NBUF, C), jnp.int32),
            pltpu.VMEM((NBUF, C + L), jnp.int32),
            pltpu.VMEM((NBUF, C, H), jnp.float32),
            pltpu.VMEM((NBUF, C, H // 2), jnp.int32),
            pltpu.VMEM((2, H // 2), jnp.int32),
            pltpu.VMEM((H // 2,), jnp.int32),
            pltpu.VMEM((H // 2,), jnp.int32),
            pltpu.SemaphoreType.DMA,
            pltpu.SemaphoreType.DMA,
            pltpu.SemaphoreType.DMA,
            pltpu.SemaphoreType.DMA,
            pltpu.SemaphoreType.DMA,
            pltpu.SemaphoreType.DMA,
        ],
        compiler_params=pltpu.CompilerParams(needs_layout_passes=False),
    )
    out = f(ids, tts, word_emb, pos_b, type_b, gam_b, bet_b)
    return out.reshape(B, S, H)


# speedup vs baseline: 1.0283x; 1.0283x over previous
import functools

import jax
import jax.numpy as jnp
from jax import lax
from jax.experimental import pallas as pl
from jax.experimental.pallas import tpu as pltpu
from jax.experimental.pallas import tpu_sc as plsc

NC, NS, L = 2, 16, 16
NW = NC * NS
C = 32
NBUF = 2


def _lane_sum(x):
    return jnp.full((L,), jnp.sum(x), dtype=jnp.float32)


def _swz(x):
    shp = x.shape
    nb = shp[-1] // (2 * L)
    y = x.reshape(*shp[:-1], nb, 2, L).swapaxes(-1, -2)
    ybf = y.reshape(*shp[:-1], shp[-1] // 2, 2).astype(jnp.bfloat16)
    return lax.bitcast_convert_type(ybf, jnp.int32)


def _unpk(v32):
    vbf = plsc.bitcast(v32, jnp.bfloat16)
    return plsc.unpack(vbf, format=plsc.PackFormat.INTERLEAVED,
                       preferred_element_type=jnp.float32)


def _sc_body(S, H, TPW, ids_hbm, tts_hbm, word_hbm, pos_hbm, type_hbm,
             gam_hbm, bet_hbm, out_hbm,
             idx_v, ttv, rows_v, pos_v, type_v, gam_v, bet_v,
             gsem0, gsem1, psem0, psem1, osem0, osem1):
    nh2 = H // (2 * L)
    nch = TPW // C
    gsem = (gsem0, gsem1)
    psem = (psem0, psem1)
    osem = (osem0, osem1)
    wid = lax.axis_index("c") * NS + lax.axis_index("s")
    tok0 = wid * TPW
    s0 = lax.rem(tok0, S)

    pltpu.sync_copy(type_hbm, type_v)
    pltpu.sync_copy(gam_hbm, gam_v)
    pltpu.sync_copy(bet_hbm, bet_v)

    inv_h = jnp.float32(1.0 / H)

    def issue_in(c, b):
        base = tok0 + c * C
        pltpu.sync_copy(ids_hbm.at[pl.ds(base, C)], idx_v.at[b])
        pltpu.sync_copy(tts_hbm.at[pl.ds(base, C)], ttv.at[b, pl.ds(0, C)])
        pltpu.async_copy(pos_hbm.at[pl.ds(s0 + c * C, C)], pos_v.at[b],
                         psem[b])
        pltpu.async_copy(word_hbm.at[idx_v.at[b]], rows_v.at[b], gsem[b])

    def wait_in(c, b):
        pltpu.make_async_copy(word_hbm.at[idx_v.at[b]], rows_v.at[b],
                              gsem[b]).wait()
        pltpu.make_async_copy(pos_hbm.at[pl.ds(s0 + c * C, C)], pos_v.at[b],
                              psem[b]).wait()

    def start_out(c, b):
        base = tok0 + c * C
        pltpu.async_copy(rows_v.at[b], out_hbm.at[pl.ds(base, C)], osem[b])

    def wait_out(c, b):
        base = tok0 + c * C
        pltpu.make_async_copy(rows_v.at[b], out_hbm.at[pl.ds(base, C)],
                              osem[b]).wait()

    def compute(c, b):
        @plsc.parallel_loop(0, C, 1)
        def tok_body(j):
            ttvec = ttv[b, pl.ds(j, L)]
            tmask = ttvec[0] != 0
            nacc = 8
            acc_s = [jnp.zeros((L,), jnp.float32) for _ in range(nacc)]
            acc_q = [jnp.zeros((L,), jnp.float32) for _ in range(nacc)]
            for h in range(nh2):
                lo = pl.ds(2 * h * L, L)
                hi = pl.ds((2 * h + 1) * L, L)
                d32 = pl.ds(h * L, L)
                p_lo, p_hi = _unpk(pos_v[b, j, d32])
                t_lo, t_hi = _unpk(jnp.where(tmask, type_v[1, d32],
                                             type_v[0, d32]))
                x_lo = rows_v[b, j, lo] + p_lo + t_lo
                x_hi = rows_v[b, j, hi] + p_hi + t_hi
                k = (2 * h) % nacc
                k2 = (2 * h + 1) % nacc
                acc_s[k] = acc_s[k] + x_lo
                acc_q[k] = acc_q[k] + x_lo * x_lo
                acc_s[k2] = acc_s[k2] + x_hi
                acc_q[k2] = acc_q[k2] + x_hi * x_hi
                rows_v[b, j, lo] = x_lo
                rows_v[b, j, hi] = x_hi
            while len(acc_s) > 1:
                acc_s = [p + q for p, q in zip(acc_s[::2], acc_s[1::2])]
                acc_q = [p + q for p, q in zip(acc_q[::2], acc_q[1::2])]
            mean = _lane_sum(acc_s[0]) * inv_h
            var = _lane_sum(acc_q[0]) * inv_h - mean * mean
            v16 = var + jnp.float32(1e-12)
            bits = plsc.bitcast(v16, jnp.int32)
            bits = jnp.int32(0x5F3759DF) - (bits >> 1)
            y = plsc.bitcast(bits, jnp.float32)
            for _ in range(3):
                y = y * (jnp.float32(1.5) - jnp.float32(0.5) * v16 * y * y)
            a = y
            bb = -mean * y
            for h in range(nh2):
                lo = pl.ds(2 * h * L, L)
                hi = pl.ds((2 * h + 1) * L, L)
                d32 = pl.ds(h * L, L)
                g_lo, g_hi = _unpk(gam_v[d32])
                be_lo, be_hi = _unpk(bet_v[d32])
                x_lo = rows_v[b, j, lo]
                x_hi = rows_v[b, j, hi]
                rows_v[b, j, lo] = (x_lo * a + bb) * g_lo + be_lo
                rows_v[b, j, hi] = (x_hi * a + bb) * g_hi + be_hi

    issue_in(0, 0)

    def pair_body(cp, carry):
        for b in range(NBUF):
            c = cp * NBUF + b

            @pl.when(c + 1 < nch)
            def _prefetch():
                @pl.when(c >= 1)
                def _drain():
                    wait_out(c - 1, 1 - b)
                issue_in(c + 1, 1 - b)

            wait_in(c, b)
            compute(c, b)
            start_out(c, b)
        return carry

    lax.fori_loop(0, nch // NBUF, pair_body, 0)
    wait_out(nch - 2, 0)
    wait_out(nch - 1, 1)


def kernel(input_ids, token_type_ids, word_emb, pos_emb, type_emb,
           ln_gamma, ln_beta):
    B, S = input_ids.shape
    H = word_emb.shape[1]
    TOK = B * S
    TPW = TOK // NW

    ids = input_ids.reshape(-1).astype(jnp.int32)
    tts = token_type_ids.reshape(-1).astype(jnp.int32)
    pos_b = _swz(pos_emb)
    type_b = _swz(type_emb)
    gam_b = _swz(ln_gamma)
    bet_b = _swz(ln_beta)

    mesh = plsc.VectorSubcoreMesh(
        core_axis_name="c", subcore_axis_name="s",
        num_cores=NC, num_subcores=NS)
    f = pl.kernel(
        functools.partial(_sc_body, S, H, TPW),
        out_type=jax.ShapeDtypeStruct((TOK, H), jnp.float32),
        mesh=mesh,
        scratch_types=[
            pltpu.VMEM((NBUF, C), jnp.int32),
            pltpu.VMEM((NBUF, C + L), jnp.int32),
            pltpu.VMEM((NBUF, C, H), jnp.float32),
            pltpu.VMEM((NBUF, C, H // 2), jnp.int32),
            pltpu.VMEM((2, H // 2), jnp.int32),
            pltpu.VMEM((H // 2,), jnp.int32),
            pltpu.VMEM((H // 2,), jnp.int32),
            pltpu.SemaphoreType.DMA,
            pltpu.SemaphoreType.DMA,
            pltpu.SemaphoreType.DMA,
            pltpu.SemaphoreType.DMA,
            pltpu.SemaphoreType.DMA,
            pltpu.SemaphoreType.DMA,
        ],
        compiler_params=pltpu.CompilerParams(needs_layout_passes=False),
    )
    out = f(ids, tts, word_emb, pos_b, type_b, gam_b, bet_b)
    return out.reshape(B, S, H)
